# baseline (speedup 1.0000x reference)
import functools

import jax
import jax.numpy as jnp
from jax import lax
from jax.experimental import pallas as pl
from jax.experimental.pallas import tpu as pltpu
from jax.experimental.pallas import tpu_sc as plsc

F = 26
V = 100000
D = 32
B = 4096
H = 50
N = B * H * F
NC = 2
NS = 16
NW = NC * NS
PER_W = N // NW
L = 16
CHUNK = F * 16
NCHUNKS = PER_W // CHUNK

_mesh = plsc.VectorSubcoreMesh(core_axis_name="c", subcore_axis_name="s")


@functools.partial(
    pl.kernel,
    mesh=_mesh,
    out_type=jax.ShapeDtypeStruct((N, D), jnp.float32),
    compiler_params=pltpu.CompilerParams(use_tc_tiling_on_sc=False),
    scratch_types=[
        pltpu.VMEM((4, CHUNK), jnp.int32),
        pltpu.VMEM((CHUNK,), jnp.int32),
        pltpu.VMEM((4, CHUNK, D), jnp.float32),
        pltpu.SemaphoreType.DMA,
        pltpu.SemaphoreType.DMA,
        pltpu.SemaphoreType.DMA,
    ],
)
def _embed(tab_hbm, idx_hbm, out_hbm, idx_v, off_v, rows_v, sem_i, sem_g, sem_s):
    wid = lax.axis_index("s") * NC + lax.axis_index("c")
    base = wid * PER_W

    for s in range(CHUNK // L):
        j = lax.iota(jnp.int32, L) + (s * L)
        off_v[pl.ds(s * L, L)] = (j % F) * V

    def idx_copy(c, k):
        return pltpu.make_async_copy(
            idx_hbm.at[pl.ds(base + c * CHUNK, CHUNK)], idx_v.at[k], sem_i)

    def gather_copy(c, k):
        return pltpu.make_async_copy(
            tab_hbm.at[idx_v.at[k]], rows_v.at[k], sem_g)

    def store_copy(c, k):
        return pltpu.make_async_copy(
            rows_v.at[k], out_hbm.at[pl.ds(base + c * CHUNK, CHUNK)], sem_s)

    def prep(c, k):
        idx_copy(c, k).wait()
        for s in range(CHUNK // L):
            sl = pl.ds(s * L, L)
            idx_v[k, sl] = idx_v[k, sl] + off_v[sl]

    def step(c, k, fire_idx, do_next, wait_store):
        gather_copy(c, k).wait()
        store_copy(c, k).start()
        if fire_idx:
            idx_copy(c + 3, (k + 3) % 4).start()
        if do_next:
            prep(c + 2, (k + 2) % 4)
        if wait_store:
            store_copy(c - 2, (k + 2) % 4).wait()
        if do_next:
            gather_copy(c + 2, (k + 2) % 4).start()

    idx_copy(0, 0).start()
    prep(0, 0)
    idx_copy(1, 1).start()
    gather_copy(0, 0).start()
    prep(1, 1)
    idx_copy(2, 2).start()
    gather_copy(1, 1).start()

    step(0, 0, fire_idx=True, do_next=True, wait_store=False)
    step(1, 1, fire_idx=True, do_next=True, wait_store=False)
    step(2, 2, fire_idx=True, do_next=True, wait_store=True)
    step(3, 3, fire_idx=True, do_next=True, wait_store=True)

    def body(t, carry):
        c = 4 * t + 4
        step(c + 0, 0, fire_idx=True, do_next=True, wait_store=True)
        step(c + 1, 1, fire_idx=True, do_next=True, wait_store=True)
        step(c + 2, 2, fire_idx=True, do_next=True, wait_store=True)
        step(c + 3, 3, fire_idx=True, do_next=True, wait_store=True)
        return carry

    lax.fori_loop(0, (NCHUNKS - 8) // 4, body, 0)

    step(NCHUNKS - 4, 0, fire_idx=True, do_next=True, wait_store=True)
    step(NCHUNKS - 3, 1, fire_idx=False, do_next=True, wait_store=True)
    step(NCHUNKS - 2, 2, fire_idx=False, do_next=False, wait_store=True)
    step(NCHUNKS - 1, 3, fire_idx=False, do_next=False, wait_store=True)
    store_copy(NCHUNKS - 2, 2).wait()
    store_copy(NCHUNKS - 1, 3).wait()


def kernel(x, tables):
    x_flat = x.reshape(-1).astype(jnp.int32)
    tab_flat = tables.reshape(F * V, D)
    out = _embed(tab_flat, x_flat)
    return out.reshape(B, H, F * D)


# speedup vs baseline: 1.0033x; 1.0033x over previous
import functools

import jax
import jax.numpy as jnp
from jax import lax
from jax.experimental import pallas as pl
from jax.experimental.pallas import tpu as pltpu
from jax.experimental.pallas import tpu_sc as plsc

F = 26
V = 100000
D = 32
B = 4096
H = 50
N = B * H * F
NC = 2
NS = 16
NW = NC * NS
PER_W = N // NW
L = 16
CHUNK = F * 32
NCHUNKS = PER_W // CHUNK

_mesh = plsc.VectorSubcoreMesh(core_axis_name="c", subcore_axis_name="s")


@functools.partial(
    pl.kernel,
    mesh=_mesh,
    out_type=jax.ShapeDtypeStruct((N, D), jnp.float32),
    compiler_params=pltpu.CompilerParams(use_tc_tiling_on_sc=False),
    scratch_types=[
        pltpu.VMEM((4, CHUNK), jnp.int32),
        pltpu.VMEM((CHUNK,), jnp.int32),
        pltpu.VMEM((4, CHUNK, D), jnp.float32),
        pltpu.SemaphoreType.DMA,
        pltpu.SemaphoreType.DMA,
        pltpu.SemaphoreType.DMA,
    ],
)
def _embed(tab_hbm, idx_hbm, out_hbm, idx_v, off_v, rows_v, sem_i, sem_g, sem_s):
    wid = lax.axis_index("s") * NC + lax.axis_index("c")
    base = wid * PER_W

    for s in range(CHUNK // L):
        j = lax.iota(jnp.int32, L) + (s * L)
        off_v[pl.ds(s * L, L)] = (j % F) * V

    def idx_copy(c, k):
        return pltpu.make_async_copy(
            idx_hbm.at[pl.ds(base + c * CHUNK, CHUNK)], idx_v.at[k], sem_i)

    def gather_copy(c, k):
        return pltpu.make_async_copy(
            tab_hbm.at[idx_v.at[k]], rows_v.at[k], sem_g)

    def store_copy(c, k):
        return pltpu.make_async_copy(
            rows_v.at[k], out_hbm.at[pl.ds(base + c * CHUNK, CHUNK)], sem_s)

    def prep(c, k):
        idx_copy(c, k).wait()
        for s in range(CHUNK // L):
            sl = pl.ds(s * L, L)
            idx_v[k, sl] = idx_v[k, sl] + off_v[sl]

    def step(c, k, fire_idx, do_next, wait_store):
        gather_copy(c, k).wait()
        store_copy(c, k).start()
        if fire_idx:
            idx_copy(c + 3, (k + 3) % 4).start()
        if do_next:
            prep(c + 2, (k + 2) % 4)
        if wait_store:
            store_copy(c - 2, (k + 2) % 4).wait()
        if do_next:
            gather_copy(c + 2, (k + 2) % 4).start()

    idx_copy(0, 0).start()
    prep(0, 0)
    idx_copy(1, 1).start()
    gather_copy(0, 0).start()
    prep(1, 1)
    idx_copy(2, 2).start()
    gather_copy(1, 1).start()

    step(0, 0, fire_idx=True, do_next=True, wait_store=False)
    step(1, 1, fire_idx=True, do_next=True, wait_store=False)
    step(2, 2, fire_idx=True, do_next=True, wait_store=True)
    step(3, 3, fire_idx=True, do_next=True, wait_store=True)

    def body(t, carry):
        c = 4 * t + 4
        step(c + 0, 0, fire_idx=True, do_next=True, wait_store=True)
        step(c + 1, 1, fire_idx=True, do_next=True, wait_store=True)
        step(c + 2, 2, fire_idx=True, do_next=True, wait_store=True)
        step(c + 3, 3, fire_idx=True, do_next=True, wait_store=True)
        return carry

    lax.fori_loop(0, (NCHUNKS - 8) // 4, body, 0)

    step(NCHUNKS - 4, 0, fire_idx=True, do_next=True, wait_store=True)
    step(NCHUNKS - 3, 1, fire_idx=False, do_next=True, wait_store=True)
    step(NCHUNKS - 2, 2, fire_idx=False, do_next=False, wait_store=True)
    step(NCHUNKS - 1, 3, fire_idx=False, do_next=False, wait_store=True)
    store_copy(NCHUNKS - 2, 2).wait()
    store_copy(NCHUNKS - 1, 3).wait()


def kernel(x, tables):
    x_flat = x.reshape(-1).astype(jnp.int32)
    tab_flat = tables.reshape(F * V, D)
    out = _embed(tab_flat, x_flat)
    return out.reshape(B, H, F * D)
